# initial kernel scaffold (unmeasured)
import jax
import jax.numpy as jnp
from jax import lax
from jax.experimental import pallas as pl
from jax.experimental.pallas import tpu as pltpu

N_DEV = 8
SCALE = 0.08838834764831843


def _virt(p):
    return p ^ ((p >> 1) & 1)


def kernel(x, Wq, Wo, K_ext, V_ext):
    B, Sq, D = x.shape
    Dh = K_ext.shape[3]
    Hl = K_ext.shape[2]
    Skv = K_ext.shape[1]
    Dl = Wq.shape[1]

    def attn_body(x_ref, wq_ref, k_ref, v_ref, o_ref):
        xb = x_ref[0].astype(jnp.bfloat16)
        q = jnp.dot(xb, wq_ref[...].astype(jnp.bfloat16),
                    preferred_element_type=jnp.float32)
        k = k_ref[0, :, 0, :].astype(jnp.bfloat16)
        s = lax.dot_general(q.astype(jnp.bfloat16), k,
                            (((1,), (1,)), ((), ())),
                            preferred_element_type=jnp.float32) * SCALE
        m = jnp.max(s, axis=1, keepdims=True)
        pexp = jnp.exp(s - m)
        l = jnp.sum(pexp, axis=1, keepdims=True)
        v = v_ref[0, :, 0, :].astype(jnp.bfloat16)
        o = jnp.dot(pexp.astype(jnp.bfloat16), v,
                    preferred_element_type=jnp.float32) / l
        o_ref[0, :, 0, :] = o

    attn = pl.pallas_call(
        attn_body,
        grid=(B, Hl),
        in_specs=[
            pl.BlockSpec((1, Sq, D), lambda b, h: (b, 0, 0)),
            pl.BlockSpec((D, Dh), lambda b, h: (0, h)),
            pl.BlockSpec((1, Skv, 1, Dh), lambda b, h: (b, 0, h, 0)),
            pl.BlockSpec((1, Skv, 1, Dh), lambda b, h: (b, 0, h, 0)),
        ],
        out_specs=pl.BlockSpec((1, Sq, 1, Dh), lambda b, h: (b, 0, h, 0)),
        out_shape=jax.ShapeDtypeStruct((B, Sq, Hl, Dh), jnp.float32),
    )(x, Wq, K_ext, V_ext)

    attn_flat = attn.reshape(B * Sq, Hl * Dh)

    R = B * Sq
    CH = R // N_DEV

    def ar_body(a_ref, wo_ref, out_ref, rs_ref, send_sems, recv_sems):
        out_ref[...] = jnp.dot(
            a_ref[...].astype(jnp.bfloat16),
            wo_ref[...].astype(jnp.bfloat16),
            preferred_element_type=jnp.float32,
        )

        p = lax.axis_index("i")
        r = _virt(p)

        lo = jnp.int32(0)
        rs_off = 0
        sz = N_DEV
        for si, msk in enumerate((4, 2, 1)):
            half_rows = (sz // 2) * CH
            partner = _virt(r ^ msk)
            keep_hi = (r & msk) != 0
            send_row = lo + jnp.where(keep_hi, 0, half_rows)
            keep_row = lo + jnp.where(keep_hi, half_rows, 0)
            rdma = pltpu.make_async_remote_copy(
                src_ref=out_ref.at[pl.ds(send_row, half_rows), :],
                dst_ref=rs_ref.at[pl.ds(rs_off, half_rows), :],
                send_sem=send_sems.at[si],
                recv_sem=recv_sems.at[si],
                device_id=(partner,),
                device_id_type=pl.DeviceIdType.MESH,
            )
            rdma.start()
            rdma.wait()
            out_ref[pl.ds(keep_row, half_rows), :] = (
                out_ref[pl.ds(keep_row, half_rows), :]
                + rs_ref[pl.ds(rs_off, half_rows), :]
            )
            lo = keep_row
            rs_off += half_rows
            sz //= 2

        for si, msk in enumerate((1, 2, 4), start=3):
            rows = msk * CH
            start_row = (r // msk) * msk * CH
            partner = _virt(r ^ msk)
            rdma = pltpu.make_async_remote_copy(
                src_ref=out_ref.at[pl.ds(start_row, rows), :],
                dst_ref=out_ref.at[pl.ds(start_row, rows), :],
                send_sem=send_sems.at[si],
                recv_sem=recv_sems.at[si],
                device_id=(partner,),
                device_id_type=pl.DeviceIdType.MESH,
            )
            rdma.start()
            rdma.wait()

    out = pl.pallas_call(
        ar_body,
        out_shape=jax.ShapeDtypeStruct((R, D), jnp.float32),
        in_specs=[
            pl.BlockSpec(memory_space=pltpu.VMEM),
            pl.BlockSpec(memory_space=pltpu.VMEM),
        ],
        out_specs=pl.BlockSpec(memory_space=pltpu.VMEM),
        scratch_shapes=[
            pltpu.VMEM(((N_DEV - 1) * CH, D), jnp.float32),
            pltpu.SemaphoreType.DMA((6,)),
            pltpu.SemaphoreType.DMA((6,)),
        ],
    )(attn_flat, Wo)

    return out.reshape(B, Sq, D)


# baseline (device time: 167784 ns/iter reference)
import jax
import jax.numpy as jnp
from jax import lax
from jax.experimental import pallas as pl
from jax.experimental.pallas import tpu as pltpu

N_DEV = 8
SCALE = 0.08838834764831843


def _virt(p):
    return p ^ ((p >> 1) & 1)


def kernel(x, Wq, Wo, K_ext, V_ext):
    B, Sq, D = x.shape
    Dh = K_ext.shape[3]
    Hl = K_ext.shape[2]
    Skv = K_ext.shape[1]
    Dl = Wq.shape[1]

    def attn_body(x_ref, wq_ref, k_ref, v_ref, o_ref):
        xb = x_ref[0].astype(jnp.bfloat16)
        qb = jnp.dot(xb, wq_ref[...].astype(jnp.bfloat16),
                     preferred_element_type=jnp.float32)
        for h in range(Hl):
            q = qb[:, h * Dh:(h + 1) * Dh].astype(jnp.bfloat16)
            k = k_ref[0, :, h, :].astype(jnp.bfloat16)
            s = lax.dot_general(q, k, (((1,), (1,)), ((), ())),
                                preferred_element_type=jnp.float32) * SCALE
            m = jnp.max(s, axis=1, keepdims=True)
            pexp = jnp.exp(s - m)
            l = jnp.sum(pexp, axis=1, keepdims=True)
            v = v_ref[0, :, h, :].astype(jnp.bfloat16)
            o = jnp.dot(pexp.astype(jnp.bfloat16), v,
                        preferred_element_type=jnp.float32) / l
            o_ref[0, :, h, :] = o

    attn = pl.pallas_call(
        attn_body,
        grid=(B,),
        in_specs=[
            pl.BlockSpec((1, Sq, D), lambda b: (b, 0, 0)),
            pl.BlockSpec((D, Hl * Dh), lambda b: (0, 0)),
            pl.BlockSpec((1, Skv, Hl, Dh), lambda b: (b, 0, 0, 0)),
            pl.BlockSpec((1, Skv, Hl, Dh), lambda b: (b, 0, 0, 0)),
        ],
        out_specs=pl.BlockSpec((1, Sq, Hl, Dh), lambda b: (b, 0, 0, 0)),
        out_shape=jax.ShapeDtypeStruct((B, Sq, Hl, Dh), jnp.float32),
    )(x, Wq, K_ext, V_ext)

    attn_flat = attn.reshape(B * Sq, Hl * Dh)

    R = B * Sq
    CH = R // N_DEV

    def ar_body(a_ref, wo_ref, out_ref, rs_ref, send_sems, recv_sems):
        out_ref[...] = jnp.dot(
            a_ref[...].astype(jnp.bfloat16),
            wo_ref[...].astype(jnp.bfloat16),
            preferred_element_type=jnp.float32,
        )

        p = lax.axis_index("i")
        r = _virt(p)

        lo = jnp.int32(0)
        rs_off = 0
        sz = N_DEV
        for si, msk in enumerate((4, 2, 1)):
            half_rows = (sz // 2) * CH
            partner = _virt(r ^ msk)
            keep_hi = (r & msk) != 0
            send_row = lo + jnp.where(keep_hi, 0, half_rows)
            keep_row = lo + jnp.where(keep_hi, half_rows, 0)
            rdma = pltpu.make_async_remote_copy(
                src_ref=out_ref.at[pl.ds(send_row, half_rows), :],
                dst_ref=rs_ref.at[pl.ds(rs_off, half_rows), :],
                send_sem=send_sems.at[si],
                recv_sem=recv_sems.at[si],
                device_id=(partner,),
                device_id_type=pl.DeviceIdType.MESH,
            )
            rdma.start()
            rdma.wait()
            out_ref[pl.ds(keep_row, half_rows), :] = (
                out_ref[pl.ds(keep_row, half_rows), :]
                + rs_ref[pl.ds(rs_off, half_rows), :]
            )
            lo = keep_row
            rs_off += half_rows
            sz //= 2

        for si, msk in enumerate((1, 2, 4), start=3):
            rows = msk * CH
            start_row = (r // msk) * msk * CH
            partner = _virt(r ^ msk)
            rdma = pltpu.make_async_remote_copy(
                src_ref=out_ref.at[pl.ds(start_row, rows), :],
                dst_ref=out_ref.at[pl.ds(start_row, rows), :],
                send_sem=send_sems.at[si],
                recv_sem=recv_sems.at[si],
                device_id=(partner,),
                device_id_type=pl.DeviceIdType.MESH,
            )
            rdma.start()
            rdma.wait()

    out = pl.pallas_call(
        ar_body,
        out_shape=jax.ShapeDtypeStruct((R, D), jnp.float32),
        in_specs=[
            pl.BlockSpec(memory_space=pltpu.VMEM),
            pl.BlockSpec(memory_space=pltpu.VMEM),
        ],
        out_specs=pl.BlockSpec(memory_space=pltpu.VMEM),
        scratch_shapes=[
            pltpu.VMEM(((N_DEV - 1) * CH, D), jnp.float32),
            pltpu.SemaphoreType.DMA((6,)),
            pltpu.SemaphoreType.DMA((6,)),
        ],
    )(attn_flat, Wo)

    return out.reshape(B, Sq, D)


# device time: 101652 ns/iter; 1.6506x vs baseline; 1.6506x over previous
import jax
import jax.numpy as jnp
from jax import lax
from jax.experimental import pallas as pl
from jax.experimental.pallas import tpu as pltpu

N_DEV = 8
SCALE = 0.08838834764831843


def _virt(p):
    return p ^ ((p >> 1) & 1)


def kernel(x, Wq, Wo, K_ext, V_ext):
    B, Sq, D = x.shape
    Dh = K_ext.shape[3]
    Hl = K_ext.shape[2]
    Skv = K_ext.shape[1]
    Dl = Wq.shape[1]

    def attn_body(x_ref, wq_ref, k_ref, v_ref, o_ref):
        xb = x_ref[0].astype(jnp.bfloat16)
        qb = jnp.dot(xb, wq_ref[...].astype(jnp.bfloat16),
                     preferred_element_type=jnp.float32)
        for h in range(Hl):
            q = qb[:, h * Dh:(h + 1) * Dh].astype(jnp.bfloat16)
            k = k_ref[0, :, h, :].astype(jnp.bfloat16)
            s = lax.dot_general(q, k, (((1,), (1,)), ((), ())),
                                preferred_element_type=jnp.float32) * SCALE
            m = jnp.max(s, axis=1, keepdims=True)
            pexp = jnp.exp(s - m)
            l = jnp.sum(pexp, axis=1, keepdims=True)
            v = v_ref[0, :, h, :].astype(jnp.bfloat16)
            o = jnp.dot(pexp.astype(jnp.bfloat16), v,
                        preferred_element_type=jnp.float32) / l
            o_ref[0, :, h, :] = o

    attn = pl.pallas_call(
        attn_body,
        grid=(B,),
        in_specs=[
            pl.BlockSpec((1, Sq, D), lambda b: (b, 0, 0)),
            pl.BlockSpec((D, Hl * Dh), lambda b: (0, 0)),
            pl.BlockSpec((1, Skv, Hl, Dh), lambda b: (b, 0, 0, 0)),
            pl.BlockSpec((1, Skv, Hl, Dh), lambda b: (b, 0, 0, 0)),
        ],
        out_specs=pl.BlockSpec((1, Sq, Hl, Dh), lambda b: (b, 0, 0, 0)),
        out_shape=jax.ShapeDtypeStruct((B, Sq, Hl, Dh), jnp.float32),
    )(x, Wq, K_ext, V_ext)

    attn_flat = attn.reshape(B * Sq, Hl * Dh)

    R = B * Sq
    GROUPS = ((0, 384), (384, 384), (768, 256))
    ORDERS = ((4, 2, 1), (2, 1, 4), (1, 4, 2))
    NG = len(GROUPS)

    rs_off = []
    off = 0
    for g, (_, nr) in enumerate(GROUPS):
        per = []
        for si in range(3):
            per.append(off)
            off += nr >> (si + 1)
        rs_off.append(per)
    RS_ROWS = off

    def ar_body(a_ref, wo_ref, out_ref, acc_ref, rs_ref, send_sems, recv_sems):
        acc_ref[...] = jnp.dot(
            a_ref[...].astype(jnp.bfloat16),
            wo_ref[...].astype(jnp.bfloat16),
            preferred_element_type=jnp.float32,
        ).astype(jnp.bfloat16)

        p = lax.axis_index("i")
        r = _virt(p)

        barrier = pltpu.get_barrier_semaphore()
        for msk in (1, 2, 4):
            pl.semaphore_signal(
                barrier, inc=1,
                device_id=(_virt(r ^ msk),),
                device_id_type=pl.DeviceIdType.MESH,
            )
        pl.semaphore_wait(barrier, NG)

        los = [jnp.int32(r0) for r0, _ in GROUPS]
        for si in range(3):
            pending = []
            for g, (_, nr) in enumerate(GROUPS):
                msk = ORDERS[g][si]
                half = nr >> (si + 1)
                partner = _virt(r ^ msk)
                keep_hi = (r & msk) != 0
                send_row = los[g] + jnp.where(keep_hi, 0, half)
                keep_row = los[g] + jnp.where(keep_hi, half, 0)
                rdma = pltpu.make_async_remote_copy(
                    src_ref=acc_ref.at[pl.ds(send_row, half), :],
                    dst_ref=rs_ref.at[pl.ds(rs_off[g][si], half), :],
                    send_sem=send_sems.at[g, si],
                    recv_sem=recv_sems.at[g, si],
                    device_id=(partner,),
                    device_id_type=pl.DeviceIdType.MESH,
                )
                rdma.start()
                pending.append((rdma, keep_row, half, rs_off[g][si]))
                los[g] = keep_row
            for rdma, keep_row, half, o in pending:
                rdma.wait()
                acc_ref[pl.ds(keep_row, half), :] = (
                    acc_ref[pl.ds(keep_row, half), :]
                    + rs_ref[pl.ds(o, half), :]
                )

        curs = [nr >> 3 for _, nr in GROUPS]
        for si in range(3):
            pending = []
            for g in range(NG):
                msk = ORDERS[g][2 - si]
                cur = curs[g]
                partner = _virt(r ^ msk)
                rdma = pltpu.make_async_remote_copy(
                    src_ref=acc_ref.at[pl.ds(los[g], cur), :],
                    dst_ref=acc_ref.at[pl.ds(los[g], cur), :],
                    send_sem=send_sems.at[g, 3 + si],
                    recv_sem=recv_sems.at[g, 3 + si],
                    device_id=(partner,),
                    device_id_type=pl.DeviceIdType.MESH,
                )
                rdma.start()
                pending.append(rdma)
                los[g] = los[g] - jnp.where((r & msk) != 0, cur, 0)
                curs[g] = 2 * cur
            for rdma in pending:
                rdma.wait()

        out_ref[...] = acc_ref[...].astype(jnp.float32)

    out = pl.pallas_call(
        ar_body,
        out_shape=jax.ShapeDtypeStruct((R, D), jnp.float32),
        in_specs=[
            pl.BlockSpec(memory_space=pltpu.VMEM),
            pl.BlockSpec(memory_space=pltpu.VMEM),
        ],
        out_specs=pl.BlockSpec(memory_space=pltpu.VMEM),
        scratch_shapes=[
            pltpu.VMEM((R, D), jnp.bfloat16),
            pltpu.VMEM((RS_ROWS, D), jnp.bfloat16),
            pltpu.SemaphoreType.DMA((NG, 6)),
            pltpu.SemaphoreType.DMA((NG, 6)),
        ],
        compiler_params=pltpu.CompilerParams(collective_id=0),
    )(attn_flat, Wo)

    return out.reshape(B, Sq, D)


# device time: 75631 ns/iter; 2.2185x vs baseline; 1.3441x over previous
import jax
import jax.numpy as jnp
from jax import lax
from jax.experimental import pallas as pl
from jax.experimental.pallas import tpu as pltpu

N_DEV = 8
SCALE = 0.08838834764831843


def _virt(p):
    return p ^ ((p >> 1) & 1)


def kernel(x, Wq, Wo, K_ext, V_ext):
    B, Sq, D = x.shape
    Dh = K_ext.shape[3]
    Hl = K_ext.shape[2]
    Skv = K_ext.shape[1]
    Dl = Wq.shape[1]

    x_bf = x.astype(jnp.bfloat16)
    wq_bf = (Wq * SCALE).astype(jnp.bfloat16)
    k_flat = K_ext.reshape(B, Skv, Hl * Dh).astype(jnp.bfloat16)
    v_flat = V_ext.reshape(B, Skv, Hl * Dh).astype(jnp.bfloat16)

    def attn_body(x_ref, wq_ref, k_ref, v_ref, o_ref):
        qb = jnp.dot(x_ref[0], wq_ref[...],
                     preferred_element_type=jnp.float32
                     ).astype(jnp.bfloat16)
        for h in range(Hl):
            cols = slice(h * Dh, (h + 1) * Dh)
            q = qb[:, cols]
            k = k_ref[0][:, cols]
            s = lax.dot_general(q, k, (((1,), (1,)), ((), ())),
                                preferred_element_type=jnp.float32)
            pexp = jnp.exp(s.astype(jnp.bfloat16))
            l = jnp.sum(pexp.astype(jnp.float32), axis=1, keepdims=True)
            o = jnp.dot(pexp, v_ref[0][:, cols],
                        preferred_element_type=jnp.float32) * (1.0 / l)
            o_ref[0, :, cols] = o.astype(jnp.bfloat16)

    attn = pl.pallas_call(
        attn_body,
        grid=(B,),
        in_specs=[
            pl.BlockSpec((1, Sq, D), lambda b: (b, 0, 0)),
            pl.BlockSpec((D, Hl * Dh), lambda b: (0, 0)),
            pl.BlockSpec((1, Skv, Hl * Dh), lambda b: (b, 0, 0)),
            pl.BlockSpec((1, Skv, Hl * Dh), lambda b: (b, 0, 0)),
        ],
        out_specs=pl.BlockSpec((1, Sq, Hl * Dh), lambda b: (b, 0, 0)),
        out_shape=jax.ShapeDtypeStruct((B, Sq, Hl * Dh), jnp.bfloat16),
    )(x_bf, wq_bf, k_flat, v_flat)

    attn_flat = attn.reshape(B * Sq, Hl * Dh)

    R = B * Sq
    GROUPS = ((0, 384), (384, 384), (768, 256))
    ORDERS = ((4, 2, 1), (2, 1, 4), (1, 4, 2))
    NG = len(GROUPS)

    rs_off = []
    off = 0
    for g, (_, nr) in enumerate(GROUPS):
        per = []
        for si in range(3):
            per.append(off)
            off += nr >> (si + 1)
        rs_off.append(per)
    RS_ROWS = off

    def ar_body(a_ref, wo_ref, out_ref, acc_ref, rs_ref, send_sems, recv_sems):
        acc_ref[...] = jnp.dot(
            a_ref[...],
            wo_ref[...].astype(jnp.bfloat16),
            preferred_element_type=jnp.float32,
        ).astype(jnp.bfloat16)

        p = lax.axis_index("i")
        r = _virt(p)

        barrier = pltpu.get_barrier_semaphore()
        for msk in (1, 2, 4):
            pl.semaphore_signal(
                barrier, inc=1,
                device_id=(_virt(r ^ msk),),
                device_id_type=pl.DeviceIdType.MESH,
            )
        pl.semaphore_wait(barrier, NG)

        los = [jnp.int32(r0) for r0, _ in GROUPS]
        for si in range(3):
            pending = []
            for g, (_, nr) in enumerate(GROUPS):
                msk = ORDERS[g][si]
                half = nr >> (si + 1)
                partner = _virt(r ^ msk)
                keep_hi = (r & msk) != 0
                send_row = los[g] + jnp.where(keep_hi, 0, half)
                keep_row = los[g] + jnp.where(keep_hi, half, 0)
                rdma = pltpu.make_async_remote_copy(
                    src_ref=acc_ref.at[pl.ds(send_row, half), :],
                    dst_ref=rs_ref.at[pl.ds(rs_off[g][si], half), :],
                    send_sem=send_sems.at[g, si],
                    recv_sem=recv_sems.at[g, si],
                    device_id=(partner,),
                    device_id_type=pl.DeviceIdType.MESH,
                )
                rdma.start()
                pending.append((rdma, keep_row, half, rs_off[g][si]))
                los[g] = keep_row
            for rdma, keep_row, half, o in pending:
                rdma.wait()
                acc_ref[pl.ds(keep_row, half), :] = (
                    acc_ref[pl.ds(keep_row, half), :]
                    + rs_ref[pl.ds(o, half), :]
                )

        curs = [nr >> 3 for _, nr in GROUPS]
        for si in range(3):
            pending = []
            for g in range(NG):
                msk = ORDERS[g][2 - si]
                cur = curs[g]
                partner = _virt(r ^ msk)
                rdma = pltpu.make_async_remote_copy(
                    src_ref=acc_ref.at[pl.ds(los[g], cur), :],
                    dst_ref=acc_ref.at[pl.ds(los[g], cur), :],
                    send_sem=send_sems.at[g, 3 + si],
                    recv_sem=recv_sems.at[g, 3 + si],
                    device_id=(partner,),
                    device_id_type=pl.DeviceIdType.MESH,
                )
                rdma.start()
                pending.append(rdma)
                los[g] = los[g] - jnp.where((r & msk) != 0, cur, 0)
                curs[g] = 2 * cur
            for rdma in pending:
                rdma.wait()

        out_ref[...] = acc_ref[...].astype(jnp.float32)

    out = pl.pallas_call(
        ar_body,
        out_shape=jax.ShapeDtypeStruct((R, D), jnp.float32),
        in_specs=[
            pl.BlockSpec(memory_space=pltpu.VMEM),
            pl.BlockSpec(memory_space=pltpu.VMEM),
        ],
        out_specs=pl.BlockSpec(memory_space=pltpu.VMEM),
        scratch_shapes=[
            pltpu.VMEM((R, D), jnp.bfloat16),
            pltpu.VMEM((RS_ROWS, D), jnp.bfloat16),
            pltpu.SemaphoreType.DMA((NG, 6)),
            pltpu.SemaphoreType.DMA((NG, 6)),
        ],
        compiler_params=pltpu.CompilerParams(collective_id=0),
    )(attn_flat, Wo)

    return out.reshape(B, Sq, D)
